# baseline (speedup 1.0000x reference)
import functools
from math import comb

import jax
import jax.numpy as jnp
import numpy as np
from jax import lax
from jax.experimental import pallas as pl
from jax.experimental.pallas import tpu as pltpu
from jax.experimental.pallas import tpu_sc as plsc

N = 10000
E = 320000
K = 10
C = 64

NC = 2
NS = 16
NW = NC * NS
W = 128
NWIN = 80
EPT = NWIN * W
E_PAD = EPT * NW
N_PAD = 10112
STRIPE = N_PAD // NS
NBUF = 8
PREF = 4
R2 = N_PAD // 2


def _mesh():
    return plsc.VectorSubcoreMesh(core_axis_name="c", subcore_axis_name="s")


def _sc_edge_pass(s_idx, g_idx, x):
    gather = g_idx is not None

    def _prologue(sid, idx_refs, rows_v, y_sp, fill_ones):
        def zrow(i, _):
            rows_v[0, i] = jnp.zeros((C,), jnp.float32)
            return 0
        lax.fori_loop(0, W, zrow, 0)
        done = 0
        while done < STRIPE:
            cnt = min(W, STRIPE - done)
            pltpu.sync_copy(rows_v.at[0, pl.ds(0, cnt)],
                            y_sp.at[pl.ds(sid * STRIPE + done, cnt)])
            done += cnt
        if fill_ones:
            def orow(i, _):
                rows_v[0, i] = jnp.ones((C,), jnp.float32)
                return 0
            lax.fori_loop(0, W, orow, 0)
        for hbm_ref, vmem_ref in idx_refs:
            pltpu.sync_copy(hbm_ref, vmem_ref)
        plsc.subcore_barrier()

    def _epilogue(cid, sid, y_sp, out_hbm):
        plsc.subcore_barrier()
        pltpu.sync_copy(y_sp.at[pl.ds(sid * STRIPE, STRIPE)],
                        out_hbm.at[cid, pl.ds(sid * STRIPE, STRIPE)])

    def body_gather(s_idx_hbm, g_idx_hbm, x_hbm, out_hbm,
                    idx_s_v, idx_g_v, rows_v, y_sp, *sems):
        gsem = sems[:NBUF]
        ssem = sems[NBUF:]
        cid = lax.axis_index("c")
        sid = lax.axis_index("s")
        wid = cid * NS + sid
        _prologue(sid, [(s_idx_hbm.at[wid], idx_s_v),
                        (g_idx_hbm.at[wid], idx_g_v)], rows_v, y_sp, False)

        def gwait(i, b):
            pltpu.make_async_copy(x_hbm.at[idx_g_v.at[i]],
                                  rows_v.at[b], gsem[b]).wait()

        def swait(i, b):
            pltpu.make_async_copy(rows_v.at[b], y_sp.at[idx_s_v.at[i]],
                                  ssem[b]).wait()

        for i in range(PREF):
            pltpu.async_copy(x_hbm.at[idx_g_v.at[i]], rows_v.at[i % NBUF],
                             gsem[i % NBUF])

        def outer(j, _):
            base = j * NBUF
            for bo in range(NBUF):
                i = base + bo
                b = bo
                gwait(i, b)
                pltpu.async_copy(rows_v.at[b], y_sp.at[idx_s_v.at[i]],
                                 ssem[b], add=True)
                nxt = i + PREF
                b2 = (bo + PREF) % NBUF

                @pl.when(nxt < NWIN)
                def _():
                    @pl.when(nxt >= NBUF)
                    def _():
                        swait(nxt - NBUF, b2)
                    pltpu.async_copy(x_hbm.at[idx_g_v.at[nxt]],
                                     rows_v.at[b2], gsem[b2])
            return 0
        lax.fori_loop(0, NWIN // NBUF, outer, 0)
        for i in range(NWIN - NBUF, NWIN):
            swait(i, i % NBUF)
        _epilogue(cid, sid, y_sp, out_hbm)

    def body_ones(s_idx_hbm, out_hbm, idx_s_v, rows_v, y_sp):
        cid = lax.axis_index("c")
        sid = lax.axis_index("s")
        wid = cid * NS + sid
        _prologue(sid, [(s_idx_hbm.at[wid], idx_s_v)], rows_v, y_sp, True)

        def win(i, _):
            pltpu.sync_copy(rows_v.at[0], y_sp.at[idx_s_v.at[i]], add=True)
            return 0
        lax.fori_loop(0, NWIN, win, 0)
        _epilogue(cid, sid, y_sp, out_hbm)

    scratch = [pltpu.VMEM((NWIN, W), jnp.int32)]
    if gather:
        scratch.append(pltpu.VMEM((NWIN, W), jnp.int32))
        scratch.append(pltpu.VMEM((NBUF, W, C), jnp.float32))
        scratch.append(pltpu.VMEM_SHARED((N_PAD, C), jnp.float32))
        scratch.extend([pltpu.SemaphoreType.DMA] * (2 * NBUF))
    else:
        scratch.append(pltpu.VMEM((1, W, C), jnp.float32))
        scratch.append(pltpu.VMEM_SHARED((N_PAD, C), jnp.float32))
    body = body_gather if gather else body_ones

    kern = pl.kernel(
        body,
        out_type=jax.ShapeDtypeStruct((NC, N_PAD, C), jnp.float32),
        mesh=_mesh(),
        scratch_types=scratch,
        compiler_params=pltpu.CompilerParams(use_tc_tiling_on_sc=False),
        name="sc_edge_pass" if gather else "sc_degree_pass",
    )
    if gather:
        return kern(s_idx, g_idx, x)
    return kern(s_idx)


MLP_RB = 632


def _mlp_body(f_ref, w1_ref, b1_ref, w2_ref, b2_ref, o_ref):
    for half in (0, 1):
        f = f_ref[:, half * 128:(half + 1) * 128]
        h = jnp.dot(f, w1_ref[...], preferred_element_type=jnp.float32)
        h = jnp.maximum(h + b1_ref[...], 0.0)
        o = jnp.dot(h, w2_ref[...], preferred_element_type=jnp.float32)
        o_ref[:, half * C:(half + 1) * C] = o + b2_ref[...]


def _mlp(features2, W1, b1, W2, b2):
    nin = W1.shape[0]
    nh = W1.shape[1]
    return pl.pallas_call(
        _mlp_body,
        grid=(R2 // MLP_RB,),
        in_specs=[
            pl.BlockSpec((MLP_RB, 2 * nin), lambda i: (i, 0)),
            pl.BlockSpec((nin, nh), lambda i: (0, 0)),
            pl.BlockSpec((1, nh), lambda i: (0, 0)),
            pl.BlockSpec((nh, C), lambda i: (0, 0)),
            pl.BlockSpec((1, C), lambda i: (0, 0)),
        ],
        out_specs=pl.BlockSpec((MLP_RB, 2 * C), lambda i: (i, 0)),
        out_shape=jax.ShapeDtypeStruct((R2, 2 * C), jnp.float32),
    )(features2, W1, b1.reshape(1, nh), W2, b2.reshape(1, C))


RB = 632


def _dinv_body(degp_ref, o_ref):
    deg = degp_ref[0] + degp_ref[1]
    o_ref[...] = jnp.where(deg > 0.0, lax.rsqrt(jnp.maximum(deg, 1e-30)), 0.0)


def _dinv(degp2):
    return pl.pallas_call(
        _dinv_body,
        grid=(R2 // RB,),
        in_specs=[pl.BlockSpec((NC, RB, 2 * C), lambda i: (0, i, 0))],
        out_specs=pl.BlockSpec((RB, 2 * C), lambda i: (i, 0)),
        out_shape=jax.ShapeDtypeStruct((R2, 2 * C), jnp.float32),
    )(degp2)


def _init_body(b_ref, x_ref, dinv_ref, acc_ref, u_ref):
    acc = b_ref[K] * x_ref[...]
    acc_ref[...] = acc
    u_ref[...] = dinv_ref[...] * acc


def _horner_init(b, x, dinv):
    return pl.pallas_call(
        _init_body,
        grid=(R2 // RB,),
        in_specs=[
            pl.BlockSpec(memory_space=pltpu.SMEM),
            pl.BlockSpec((RB, 2 * C), lambda i: (i, 0)),
            pl.BlockSpec((RB, 2 * C), lambda i: (i, 0)),
        ],
        out_specs=[
            pl.BlockSpec((RB, 2 * C), lambda i: (i, 0)),
            pl.BlockSpec((RB, 2 * C), lambda i: (i, 0)),
        ],
        out_shape=[
            jax.ShapeDtypeStruct((R2, 2 * C), jnp.float32),
            jax.ShapeDtypeStruct((R2, 2 * C), jnp.float32),
        ],
    )(b, x, dinv)


def _combine_body(m, b_ref, x_ref, acc_ref, p_ref, dinv_ref, out_ref, u_ref):
    s = p_ref[0] + p_ref[1]
    acc = b_ref[m] * x_ref[...] + acc_ref[...] - dinv_ref[...] * s
    out_ref[...] = acc
    u_ref[...] = dinv_ref[...] * acc


def _horner_step(m, b, x, acc, p2, dinv):
    return pl.pallas_call(
        functools.partial(_combine_body, m),
        grid=(R2 // RB,),
        in_specs=[
            pl.BlockSpec(memory_space=pltpu.SMEM),
            pl.BlockSpec((RB, 2 * C), lambda i: (i, 0)),
            pl.BlockSpec((RB, 2 * C), lambda i: (i, 0)),
            pl.BlockSpec((NC, RB, 2 * C), lambda i: (0, i, 0)),
            pl.BlockSpec((RB, 2 * C), lambda i: (i, 0)),
        ],
        out_specs=[
            pl.BlockSpec((RB, 2 * C), lambda i: (i, 0)),
            pl.BlockSpec((RB, 2 * C), lambda i: (i, 0)),
        ],
        out_shape=[
            jax.ShapeDtypeStruct((R2, 2 * C), jnp.float32),
            jax.ShapeDtypeStruct((R2, 2 * C), jnp.float32),
        ],
    )(b, x, acc, p2, dinv)


_BMAT = np.zeros((K + 1, K + 1), np.float64)
for _m in range(K + 1):
    for _i in range(_m + 1):
        _BMAT[_m, _i] = comb(K, _m) * comb(_m, _i) * ((-1.0) ** (_m - _i)) / (2.0 ** _m)
_BMAT = _BMAT.astype(np.float32)


def kernel(features, edge_index, W1, b1, W2, b2, temp):
    src = edge_index[0]
    dst = edge_index[1]
    pad_n = E_PAD - E
    spread = jnp.arange(pad_n, dtype=jnp.int32)
    pad_gather = spread % N
    pad_dump = N + spread % (N_PAD - N)
    g_idx = jnp.concatenate([src, pad_gather]).reshape(NW, NWIN, W)
    s_idx = jnp.concatenate([dst, pad_dump]).reshape(NW, NWIN, W)
    d_idx = jnp.concatenate([src, pad_dump]).reshape(NW, NWIN, W)

    nin = features.shape[1]
    features_pad = jnp.zeros((N_PAD, nin), jnp.float32).at[:N].set(features)
    b = jnp.sum(jnp.asarray(_BMAT) * jax.nn.relu(temp)[None, :], axis=1)

    x = _mlp(features_pad.reshape(R2, 2 * nin), W1, b1, W2, b2)
    degp = _sc_edge_pass(d_idx, None, None)
    dinv = _dinv(degp.reshape(NC, R2, 2 * C))
    acc, u = _horner_init(b, x, dinv)
    for m in range(K - 1, -1, -1):
        p = _sc_edge_pass(s_idx, g_idx, u.reshape(N_PAD, C))
        acc, u = _horner_step(m, b, x, acc, p.reshape(NC, R2, 2 * C), dinv)
    return acc.reshape(N_PAD, C)[:N]


# speedup vs baseline: 1.0800x; 1.0800x over previous
import functools
from math import comb

import jax
import jax.numpy as jnp
import numpy as np
from jax import lax
from jax.experimental import pallas as pl
from jax.experimental.pallas import tpu as pltpu
from jax.experimental.pallas import tpu_sc as plsc

N = 10000
E = 320000
K = 10
C = 64

NC = 2
NS = 16
NW = NC * NS
W = 128
NWIN = 80
EPT = NWIN * W
E_PAD = EPT * NW
N_PAD = 10112
STRIPE = N_PAD // NS
NBUF = 4
R2 = N_PAD // 2


def _mesh():
    return plsc.VectorSubcoreMesh(core_axis_name="c", subcore_axis_name="s")


def _sc_edge_pass(s_idx, g_idx, x):
    gather = g_idx is not None

    def _prologue(sid, idx_refs, rows_v, y_sp, fill_ones):
        def zrow(i, _):
            rows_v[0, i] = jnp.zeros((C,), jnp.float32)
            return 0
        lax.fori_loop(0, W, zrow, 0)
        done = 0
        while done < STRIPE:
            cnt = min(W, STRIPE - done)
            pltpu.sync_copy(rows_v.at[0, pl.ds(0, cnt)],
                            y_sp.at[pl.ds(sid * STRIPE + done, cnt)])
            done += cnt
        if fill_ones:
            def orow(i, _):
                rows_v[0, i] = jnp.ones((C,), jnp.float32)
                return 0
            lax.fori_loop(0, W, orow, 0)
        for hbm_ref, vmem_ref in idx_refs:
            pltpu.sync_copy(hbm_ref, vmem_ref)
        plsc.subcore_barrier()

    def _epilogue(cid, sid, y_sp, out_hbm):
        plsc.subcore_barrier()
        pltpu.sync_copy(y_sp.at[pl.ds(sid * STRIPE, STRIPE)],
                        out_hbm.at[cid, pl.ds(sid * STRIPE, STRIPE)])

    def body_gather(s_idx_hbm, g_idx_hbm, x_hbm, out_hbm,
                    idx_s_v, idx_g_v, rows_v, y_sp, *sems):
        gsem = sems
        cid = lax.axis_index("c")
        sid = lax.axis_index("s")
        wid = cid * NS + sid
        _prologue(sid, [(s_idx_hbm.at[wid], idx_s_v),
                        (g_idx_hbm.at[wid], idx_g_v)], rows_v, y_sp, False)

        for b in range(NBUF):
            pltpu.async_copy(x_hbm.at[idx_g_v.at[b]], rows_v.at[b], gsem[b])

        def outer(j, _):
            base = j * NBUF
            for b in range(NBUF):
                i = base + b
                pltpu.make_async_copy(x_hbm.at[idx_g_v.at[i]],
                                      rows_v.at[b], gsem[b]).wait()
                pltpu.sync_copy(rows_v.at[b], y_sp.at[idx_s_v.at[i]],
                                add=True)

                @pl.when(i + NBUF < NWIN)
                def _():
                    pltpu.async_copy(x_hbm.at[idx_g_v.at[i + NBUF]],
                                     rows_v.at[b], gsem[b])
            return 0
        lax.fori_loop(0, NWIN // NBUF, outer, 0)
        _epilogue(cid, sid, y_sp, out_hbm)

    def body_ones(s_idx_hbm, out_hbm, idx_s_v, rows_v, y_sp):
        cid = lax.axis_index("c")
        sid = lax.axis_index("s")
        wid = cid * NS + sid
        _prologue(sid, [(s_idx_hbm.at[wid], idx_s_v)], rows_v, y_sp, True)

        def win(i, _):
            pltpu.sync_copy(rows_v.at[0], y_sp.at[idx_s_v.at[i]], add=True)
            return 0
        lax.fori_loop(0, NWIN, win, 0)
        _epilogue(cid, sid, y_sp, out_hbm)

    scratch = [pltpu.VMEM((NWIN, W), jnp.int32)]
    if gather:
        scratch.append(pltpu.VMEM((NWIN, W), jnp.int32))
        scratch.append(pltpu.VMEM((NBUF, W, C), jnp.float32))
        scratch.append(pltpu.VMEM_SHARED((N_PAD, C), jnp.float32))
        scratch.extend([pltpu.SemaphoreType.DMA] * NBUF)
    else:
        scratch.append(pltpu.VMEM((1, W, C), jnp.float32))
        scratch.append(pltpu.VMEM_SHARED((N_PAD, C), jnp.float32))
    body = body_gather if gather else body_ones

    kern = pl.kernel(
        body,
        out_type=jax.ShapeDtypeStruct((NC, N_PAD, C), jnp.float32),
        mesh=_mesh(),
        scratch_types=scratch,
        compiler_params=pltpu.CompilerParams(use_tc_tiling_on_sc=False),
        name="sc_edge_pass" if gather else "sc_degree_pass",
    )
    if gather:
        return kern(s_idx, g_idx, x)
    return kern(s_idx)


MLP_RB = 632


def _mlp_body(f_ref, w1_ref, b1_ref, w2_ref, b2_ref, o_ref):
    for half in (0, 1):
        f = f_ref[:, half * 128:(half + 1) * 128]
        h = jnp.dot(f, w1_ref[...], preferred_element_type=jnp.float32)
        h = jnp.maximum(h + b1_ref[...], 0.0)
        o = jnp.dot(h, w2_ref[...], preferred_element_type=jnp.float32)
        o_ref[:, half * C:(half + 1) * C] = o + b2_ref[...]


def _mlp(features2, W1, b1, W2, b2):
    nin = W1.shape[0]
    nh = W1.shape[1]
    return pl.pallas_call(
        _mlp_body,
        grid=(R2 // MLP_RB,),
        in_specs=[
            pl.BlockSpec((MLP_RB, 2 * nin), lambda i: (i, 0)),
            pl.BlockSpec((nin, nh), lambda i: (0, 0)),
            pl.BlockSpec((1, nh), lambda i: (0, 0)),
            pl.BlockSpec((nh, C), lambda i: (0, 0)),
            pl.BlockSpec((1, C), lambda i: (0, 0)),
        ],
        out_specs=pl.BlockSpec((MLP_RB, 2 * C), lambda i: (i, 0)),
        out_shape=jax.ShapeDtypeStruct((R2, 2 * C), jnp.float32),
    )(features2, W1, b1.reshape(1, nh), W2, b2.reshape(1, C))


RB = 632


def _dinv_body(degp_ref, o_ref):
    deg = degp_ref[0] + degp_ref[1]
    o_ref[...] = jnp.where(deg > 0.0, lax.rsqrt(jnp.maximum(deg, 1e-30)), 0.0)


def _dinv(degp2):
    return pl.pallas_call(
        _dinv_body,
        grid=(R2 // RB,),
        in_specs=[pl.BlockSpec((NC, RB, 2 * C), lambda i: (0, i, 0))],
        out_specs=pl.BlockSpec((RB, 2 * C), lambda i: (i, 0)),
        out_shape=jax.ShapeDtypeStruct((R2, 2 * C), jnp.float32),
    )(degp2)


def _init_body(b_ref, x_ref, dinv_ref, acc_ref, u_ref):
    acc = b_ref[K] * x_ref[...]
    acc_ref[...] = acc
    u_ref[...] = dinv_ref[...] * acc


def _horner_init(b, x, dinv):
    return pl.pallas_call(
        _init_body,
        grid=(R2 // RB,),
        in_specs=[
            pl.BlockSpec(memory_space=pltpu.SMEM),
            pl.BlockSpec((RB, 2 * C), lambda i: (i, 0)),
            pl.BlockSpec((RB, 2 * C), lambda i: (i, 0)),
        ],
        out_specs=[
            pl.BlockSpec((RB, 2 * C), lambda i: (i, 0)),
            pl.BlockSpec((RB, 2 * C), lambda i: (i, 0)),
        ],
        out_shape=[
            jax.ShapeDtypeStruct((R2, 2 * C), jnp.float32),
            jax.ShapeDtypeStruct((R2, 2 * C), jnp.float32),
        ],
    )(b, x, dinv)


def _combine_body(m, b_ref, x_ref, acc_ref, p_ref, dinv_ref, out_ref, u_ref):
    s = p_ref[0] + p_ref[1]
    acc = b_ref[m] * x_ref[...] + acc_ref[...] - dinv_ref[...] * s
    out_ref[...] = acc
    u_ref[...] = dinv_ref[...] * acc


def _horner_step(m, b, x, acc, p2, dinv):
    return pl.pallas_call(
        functools.partial(_combine_body, m),
        grid=(R2 // RB,),
        in_specs=[
            pl.BlockSpec(memory_space=pltpu.SMEM),
            pl.BlockSpec((RB, 2 * C), lambda i: (i, 0)),
            pl.BlockSpec((RB, 2 * C), lambda i: (i, 0)),
            pl.BlockSpec((NC, RB, 2 * C), lambda i: (0, i, 0)),
            pl.BlockSpec((RB, 2 * C), lambda i: (i, 0)),
        ],
        out_specs=[
            pl.BlockSpec((RB, 2 * C), lambda i: (i, 0)),
            pl.BlockSpec((RB, 2 * C), lambda i: (i, 0)),
        ],
        out_shape=[
            jax.ShapeDtypeStruct((R2, 2 * C), jnp.float32),
            jax.ShapeDtypeStruct((R2, 2 * C), jnp.float32),
        ],
    )(b, x, acc, p2, dinv)


_BMAT = np.zeros((K + 1, K + 1), np.float64)
for _m in range(K + 1):
    for _i in range(_m + 1):
        _BMAT[_m, _i] = comb(K, _m) * comb(_m, _i) * ((-1.0) ** (_m - _i)) / (2.0 ** _m)
_BMAT = _BMAT.astype(np.float32)


def kernel(features, edge_index, W1, b1, W2, b2, temp):
    src = edge_index[0]
    dst = edge_index[1]
    pad_n = E_PAD - E
    spread = jnp.arange(pad_n, dtype=jnp.int32)
    pad_gather = spread % N
    pad_dump = N + spread % (N_PAD - N)
    g_idx = jnp.concatenate([src, pad_gather]).reshape(NW, NWIN, W)
    s_idx = jnp.concatenate([dst, pad_dump]).reshape(NW, NWIN, W)
    d_idx = jnp.concatenate([src, pad_dump]).reshape(NW, NWIN, W)

    nin = features.shape[1]
    features_pad = jnp.zeros((N_PAD, nin), jnp.float32).at[:N].set(features)
    b = jnp.sum(jnp.asarray(_BMAT) * jax.nn.relu(temp)[None, :], axis=1)

    x = _mlp(features_pad.reshape(R2, 2 * nin), W1, b1, W2, b2)
    degp = _sc_edge_pass(d_idx, None, None)
    dinv = _dinv(degp.reshape(NC, R2, 2 * C))
    acc, u = _horner_init(b, x, dinv)
    for m in range(K - 1, -1, -1):
        p = _sc_edge_pass(s_idx, g_idx, u.reshape(N_PAD, C))
        acc, u = _horner_step(m, b, x, acc, p.reshape(NC, R2, 2 * C), dinv)
    return acc.reshape(N_PAD, C)[:N]


# speedup vs baseline: 1.1138x; 1.0313x over previous
import functools
from math import comb

import jax
import jax.numpy as jnp
import numpy as np
from jax import lax
from jax.experimental import pallas as pl
from jax.experimental.pallas import tpu as pltpu
from jax.experimental.pallas import tpu_sc as plsc

N = 10000
E = 320000
K = 10
C = 64

NC = 2
NS = 16
NW = NC * NS
W = 128
NWIN = 80
EPT = NWIN * W
E_PAD = EPT * NW
N_PAD = 10112
STRIPE = N_PAD // NS
NBUF = 4
R2 = N_PAD // 2


def _mesh():
    return plsc.VectorSubcoreMesh(core_axis_name="c", subcore_axis_name="s")


def _sc_edge_pass(s_idx, g_idx, x):
    gather = g_idx is not None

    def _prologue(sid, idx_refs, rows_v, y_sp, fill_ones):
        def zrow(i, _):
            rows_v[0, i] = jnp.zeros((C,), jnp.float32)
            return 0
        lax.fori_loop(0, W, zrow, 0)
        done = 0
        while done < STRIPE:
            cnt = min(W, STRIPE - done)
            pltpu.sync_copy(rows_v.at[0, pl.ds(0, cnt)],
                            y_sp.at[pl.ds(sid * STRIPE + done, cnt)])
            done += cnt
        if fill_ones:
            def orow(i, _):
                rows_v[0, i] = jnp.ones((C,), jnp.float32)
                return 0
            lax.fori_loop(0, W, orow, 0)
        for hbm_ref, vmem_ref in idx_refs:
            pltpu.sync_copy(hbm_ref, vmem_ref)
        plsc.subcore_barrier()

    def _epilogue(cid, sid, y_sp, out_hbm):
        plsc.subcore_barrier()
        pltpu.sync_copy(y_sp.at[pl.ds(sid * STRIPE, STRIPE)],
                        out_hbm.at[cid, pl.ds(sid * STRIPE, STRIPE)])

    def body_gather(s_idx_hbm, g_idx_hbm, x_hbm, out_hbm,
                    idx_s_v, idx_g_v, rows_v, zbuf_v, y_sp, *sems):
        gsem = sems
        cid = lax.axis_index("c")
        sid = lax.axis_index("s")
        wid = cid * NS + sid

        pltpu.sync_copy(s_idx_hbm.at[wid], idx_s_v)
        pltpu.sync_copy(g_idx_hbm.at[wid], idx_g_v)
        for b in range(NBUF):
            pltpu.async_copy(x_hbm.at[idx_g_v.at[b]], rows_v.at[b], gsem[b])

        def zrow(i, _):
            zbuf_v[i] = jnp.zeros((C,), jnp.float32)
            return 0
        lax.fori_loop(0, W, zrow, 0)
        done = 0
        while done < STRIPE:
            cnt = min(W, STRIPE - done)
            pltpu.sync_copy(zbuf_v.at[pl.ds(0, cnt)],
                            y_sp.at[pl.ds(sid * STRIPE + done, cnt)])
            done += cnt
        plsc.subcore_barrier()


        def outer(j, _):
            base = j * NBUF
            for b in range(NBUF):
                i = base + b
                pltpu.make_async_copy(x_hbm.at[idx_g_v.at[i]],
                                      rows_v.at[b], gsem[b]).wait()
                pltpu.sync_copy(rows_v.at[b], y_sp.at[idx_s_v.at[i]],
                                add=True)

                @pl.when(i + NBUF < NWIN)
                def _():
                    pltpu.async_copy(x_hbm.at[idx_g_v.at[i + NBUF]],
                                     rows_v.at[b], gsem[b])
            return 0
        lax.fori_loop(0, NWIN // NBUF, outer, 0)
        _epilogue(cid, sid, y_sp, out_hbm)

    def body_ones(s_idx_hbm, out_hbm, idx_s_v, rows_v, y_sp):
        cid = lax.axis_index("c")
        sid = lax.axis_index("s")
        wid = cid * NS + sid
        _prologue(sid, [(s_idx_hbm.at[wid], idx_s_v)], rows_v, y_sp, True)

        def win(i, _):
            pltpu.sync_copy(rows_v.at[0], y_sp.at[idx_s_v.at[i]], add=True)
            return 0
        lax.fori_loop(0, NWIN, win, 0)
        _epilogue(cid, sid, y_sp, out_hbm)

    scratch = [pltpu.VMEM((NWIN, W), jnp.int32)]
    if gather:
        scratch.append(pltpu.VMEM((NWIN, W), jnp.int32))
        scratch.append(pltpu.VMEM((NBUF, W, C), jnp.float32))
        scratch.append(pltpu.VMEM((W, C), jnp.float32))
        scratch.append(pltpu.VMEM_SHARED((N_PAD, C), jnp.float32))
        scratch.extend([pltpu.SemaphoreType.DMA] * NBUF)
    else:
        scratch.append(pltpu.VMEM((1, W, C), jnp.float32))
        scratch.append(pltpu.VMEM_SHARED((N_PAD, C), jnp.float32))
    body = body_gather if gather else body_ones

    kern = pl.kernel(
        body,
        out_type=jax.ShapeDtypeStruct((NC, N_PAD, C), jnp.float32),
        mesh=_mesh(),
        scratch_types=scratch,
        compiler_params=pltpu.CompilerParams(use_tc_tiling_on_sc=False),
        name="sc_edge_pass" if gather else "sc_degree_pass",
    )
    if gather:
        return kern(s_idx, g_idx, x)
    return kern(s_idx)


MLP_RB = 632


def _mlp_body(f_ref, w1_ref, b1_ref, w2_ref, b2_ref, o_ref):
    for half in (0, 1):
        f = f_ref[:, half * 128:(half + 1) * 128]
        h = jnp.dot(f, w1_ref[...], preferred_element_type=jnp.float32)
        h = jnp.maximum(h + b1_ref[...], 0.0)
        o = jnp.dot(h, w2_ref[...], preferred_element_type=jnp.float32)
        o_ref[:, half * C:(half + 1) * C] = o + b2_ref[...]


def _mlp(features2, W1, b1, W2, b2):
    nin = W1.shape[0]
    nh = W1.shape[1]
    return pl.pallas_call(
        _mlp_body,
        grid=(R2 // MLP_RB,),
        in_specs=[
            pl.BlockSpec((MLP_RB, 2 * nin), lambda i: (i, 0)),
            pl.BlockSpec((nin, nh), lambda i: (0, 0)),
            pl.BlockSpec((1, nh), lambda i: (0, 0)),
            pl.BlockSpec((nh, C), lambda i: (0, 0)),
            pl.BlockSpec((1, C), lambda i: (0, 0)),
        ],
        out_specs=pl.BlockSpec((MLP_RB, 2 * C), lambda i: (i, 0)),
        out_shape=jax.ShapeDtypeStruct((R2, 2 * C), jnp.float32),
    )(features2, W1, b1.reshape(1, nh), W2, b2.reshape(1, C))


RB = 632


def _dinv_body(degp_ref, o_ref):
    deg = degp_ref[0] + degp_ref[1]
    o_ref[...] = jnp.where(deg > 0.0, lax.rsqrt(jnp.maximum(deg, 1e-30)), 0.0)


def _dinv(degp2):
    return pl.pallas_call(
        _dinv_body,
        grid=(R2 // RB,),
        in_specs=[pl.BlockSpec((NC, RB, 2 * C), lambda i: (0, i, 0))],
        out_specs=pl.BlockSpec((RB, 2 * C), lambda i: (i, 0)),
        out_shape=jax.ShapeDtypeStruct((R2, 2 * C), jnp.float32),
    )(degp2)


def _init_body(b_ref, x_ref, dinv_ref, acc_ref, u_ref):
    acc = b_ref[K] * x_ref[...]
    acc_ref[...] = acc
    u_ref[...] = dinv_ref[...] * acc


def _horner_init(b, x, dinv):
    return pl.pallas_call(
        _init_body,
        grid=(R2 // RB,),
        in_specs=[
            pl.BlockSpec(memory_space=pltpu.SMEM),
            pl.BlockSpec((RB, 2 * C), lambda i: (i, 0)),
            pl.BlockSpec((RB, 2 * C), lambda i: (i, 0)),
        ],
        out_specs=[
            pl.BlockSpec((RB, 2 * C), lambda i: (i, 0)),
            pl.BlockSpec((RB, 2 * C), lambda i: (i, 0)),
        ],
        out_shape=[
            jax.ShapeDtypeStruct((R2, 2 * C), jnp.float32),
            jax.ShapeDtypeStruct((R2, 2 * C), jnp.float32),
        ],
    )(b, x, dinv)


def _combine_body(m, b_ref, x_ref, acc_ref, p_ref, dinv_ref, out_ref, u_ref):
    s = p_ref[0] + p_ref[1]
    acc = b_ref[m] * x_ref[...] + acc_ref[...] - dinv_ref[...] * s
    out_ref[...] = acc
    u_ref[...] = dinv_ref[...] * acc


def _horner_step(m, b, x, acc, p2, dinv):
    return pl.pallas_call(
        functools.partial(_combine_body, m),
        grid=(R2 // RB,),
        in_specs=[
            pl.BlockSpec(memory_space=pltpu.SMEM),
            pl.BlockSpec((RB, 2 * C), lambda i: (i, 0)),
            pl.BlockSpec((RB, 2 * C), lambda i: (i, 0)),
            pl.BlockSpec((NC, RB, 2 * C), lambda i: (0, i, 0)),
            pl.BlockSpec((RB, 2 * C), lambda i: (i, 0)),
        ],
        out_specs=[
            pl.BlockSpec((RB, 2 * C), lambda i: (i, 0)),
            pl.BlockSpec((RB, 2 * C), lambda i: (i, 0)),
        ],
        out_shape=[
            jax.ShapeDtypeStruct((R2, 2 * C), jnp.float32),
            jax.ShapeDtypeStruct((R2, 2 * C), jnp.float32),
        ],
    )(b, x, acc, p2, dinv)


_BMAT = np.zeros((K + 1, K + 1), np.float64)
for _m in range(K + 1):
    for _i in range(_m + 1):
        _BMAT[_m, _i] = comb(K, _m) * comb(_m, _i) * ((-1.0) ** (_m - _i)) / (2.0 ** _m)
_BMAT = _BMAT.astype(np.float32)


def kernel(features, edge_index, W1, b1, W2, b2, temp):
    src = edge_index[0]
    dst = edge_index[1]
    pad_n = E_PAD - E
    spread = jnp.arange(pad_n, dtype=jnp.int32)
    pad_gather = spread % N
    pad_dump = N + spread % (N_PAD - N)
    g_idx = jnp.concatenate([src, pad_gather]).reshape(NW, NWIN, W)
    s_idx = jnp.concatenate([dst, pad_dump]).reshape(NW, NWIN, W)
    d_idx = jnp.concatenate([src, pad_dump]).reshape(NW, NWIN, W)

    nin = features.shape[1]
    features_pad = jnp.zeros((N_PAD, nin), jnp.float32).at[:N].set(features)
    b = jnp.sum(jnp.asarray(_BMAT) * jax.nn.relu(temp)[None, :], axis=1)

    x = _mlp(features_pad.reshape(R2, 2 * nin), W1, b1, W2, b2)
    degp = _sc_edge_pass(d_idx, None, None)
    dinv = _dinv(degp.reshape(NC, R2, 2 * C))
    acc, u = _horner_init(b, x, dinv)
    for m in range(K - 1, -1, -1):
        p = _sc_edge_pass(s_idx, g_idx, u.reshape(N_PAD, C))
        acc, u = _horner_step(m, b, x, acc, p.reshape(NC, R2, 2 * C), dinv)
    return acc.reshape(N_PAD, C)[:N]
